# baseline (speedup 1.0000x reference)
import functools

import jax
import jax.numpy as jnp
from jax import lax
from jax.experimental import pallas as pl
from jax.experimental.pallas import tpu as pltpu
from jax.experimental.pallas import tpu_sc as plsc

EMB = 512
NSEQ = 16
MAXLEN = 3072
TOTAL = 32768
LEN_EVEN = 1024
LEN_DELTA = 2048
PAIR = 4096

NW = 32
S_PER_W = MAXLEN // NW
SUB = 32
NSUB = S_PER_W // SUB
NUNIT = NSEQ * NSUB
NBUF = 3
LANE = 16
VPR = EMB // LANE


def _unit_params(t, s0):
    u = t >> 4
    b = t & 15
    odd = b & 1
    len_b = LEN_EVEN + odd * LEN_DELTA
    off_b = ((b >> 1) << 12) + (odd << 10)
    pr = u << 5
    ss = s0 + pr
    nv = jnp.clip(len_b - ss, 0, SUB)
    return b, ss, off_b + ss, pr, nv


def _pe_pad_body(x_hbm, pe_hbm, out_hbm,
                 pe_buf, zbuf, buf0, buf1, buf2,
                 si0, si1, si2, so0, so1, so2):
    wid = lax.axis_index("s") * 2 + lax.axis_index("c")
    s0 = wid * S_PER_W

    bufs = (buf0, buf1, buf2)
    in_sems = (si0, si1, si2)
    out_sems = (so0, so1, so2)

    pltpu.sync_copy(pe_hbm.at[pl.ds(s0, S_PER_W)], pe_buf)

    zero = jnp.zeros((LANE,), jnp.float32)

    def zero_row(i, _):
        for j in range(VPR):
            zbuf[i, 0, pl.ds(j * LANE, LANE)] = zero
        return 0

    lax.fori_loop(0, SUB, zero_row, 0)

    def start_in(t, r):
        _, _, xrow, _, nv = _unit_params(t, s0)

        @pl.when(nv > 0)
        def _():
            pltpu.make_async_copy(
                x_hbm.at[pl.ds(xrow, SUB)], bufs[r], in_sems[r]).start()

    def wait_out(r, ss, b):
        pltpu.make_async_copy(
            bufs[r], out_hbm.at[pl.ds(ss, SUB), pl.ds(b, 1)],
            out_sems[r]).wait()

    start_in(0, 0)

    def group(g, _):
        for r in range(NBUF):
            t = g * NBUF + r
            b, ss, _, pr, nv = _unit_params(t, s0)

            @pl.when(nv > 0)
            def _valid():
                pltpu.make_async_copy(
                    x_hbm.at[pl.ds(0, SUB)], bufs[r], in_sems[r]).wait()

                def add_row(i, _):
                    for j in range(VPR):
                        sl = pl.ds(j * LANE, LANE)
                        bufs[r][i, 0, sl] = (
                            bufs[r][i, 0, sl] + pe_buf[pr + i, 0, sl])
                    return 0

                lax.fori_loop(0, nv, add_row, 0)

                def pad_row(i, _):
                    for j in range(VPR):
                        bufs[r][i, 0, pl.ds(j * LANE, LANE)] = zero
                    return 0

                lax.fori_loop(nv, SUB, pad_row, 0)
                pltpu.make_async_copy(
                    bufs[r], out_hbm.at[pl.ds(ss, SUB), pl.ds(b, 1)],
                    out_sems[r]).start()

            @pl.when(nv <= 0)
            def _pad_only():
                pltpu.make_async_copy(
                    zbuf, out_hbm.at[pl.ds(ss, SUB), pl.ds(b, 1)],
                    out_sems[r]).start()

            r1 = (r + 1) % NBUF
            if r1 == 0:
                @pl.when(t + 1 < NUNIT)
                def _():
                    b1, ss1, _, _, _ = _unit_params(t + 1, s0)

                    @pl.when(t + 1 >= NBUF)
                    def _():
                        wait_out(r1, ss1, b1)

                    start_in(t + 1, r1)
            else:
                b1, ss1, _, _, _ = _unit_params(t + 1, s0)

                @pl.when(t + 1 >= NBUF)
                def _():
                    wait_out(r1, ss1, b1)

                start_in(t + 1, r1)

        return 0

    lax.fori_loop(0, NUNIT // NBUF, group, 0)

    for r in range(NBUF):
        t = NUNIT - NBUF + r
        b, ss, _, _, _ = _unit_params(t, s0)
        wait_out(r, ss, b)


_row_buf = lambda n: pltpu.VMEM((n, 1, EMB), jnp.float32)

_pe_pad_kernel = functools.partial(
    pl.kernel,
    out_type=jax.ShapeDtypeStruct((MAXLEN, NSEQ, EMB), jnp.float32),
    mesh=plsc.VectorSubcoreMesh(core_axis_name="c", subcore_axis_name="s",
                                num_cores=2, num_subcores=16),
    scratch_types=[
        _row_buf(S_PER_W),
        _row_buf(SUB),
        _row_buf(SUB), _row_buf(SUB), _row_buf(SUB),
        pltpu.SemaphoreType.DMA, pltpu.SemaphoreType.DMA, pltpu.SemaphoreType.DMA,
        pltpu.SemaphoreType.DMA, pltpu.SemaphoreType.DMA, pltpu.SemaphoreType.DMA,
    ],
)(_pe_pad_body)


def kernel(x, length, pe):
    del length
    x3 = x.reshape(TOTAL, 1, EMB)
    return _pe_pad_kernel(x3, pe)


# speedup vs baseline: 2.0884x; 2.0884x over previous
import functools

import jax
import jax.numpy as jnp
from jax import lax
from jax.experimental import pallas as pl
from jax.experimental.pallas import tpu as pltpu
from jax.experimental.pallas import tpu_sc as plsc

EMB = 512
NSEQ = 16
MAXLEN = 3072
TOTAL = 32768
LEN_EVEN = 1024
PAIR = 4096

NW = 32
S_PER_W = MAXLEN // NW
SROWS = 4
NSLAB = S_PER_W // SROWS
LANE = 16
VPR = EMB // LANE

_X_OFF = [(b >> 1) * PAIR + (b & 1) * LEN_EVEN for b in range(NSEQ)]


def _pe_pad_body(x_hbm, pe_hbm, out_hbm, pe_buf, ob0, ob1,
                 si0, si1, so0, so1):
    wid = lax.axis_index("s") * 2 + lax.axis_index("c")
    s0 = wid * S_PER_W

    obs = (ob0, ob1)
    in_sems = (si0, si1)
    out_sems = (so0, so1)

    pltpu.sync_copy(pe_hbm.at[pl.ds(s0, S_PER_W)], pe_buf)

    zero = jnp.zeros((LANE,), jnp.float32)
    ev_z = jnp.maximum(s0, LEN_EVEN)

    def in_copy(par, b, xrow):
        return pltpu.make_async_copy(
            x_hbm.at[pl.ds(xrow, SROWS)],
            obs[par].at[:, pl.ds(b, 1)], in_sems[par])

    def out_copy(par, ss):
        return pltpu.make_async_copy(
            obs[par], out_hbm.at[pl.ds(ss, SROWS)], out_sems[par])

    def slab(g, _):
        for par in range(2):
            k = 2 * g + par
            pr = k * SROWS
            ss = s0 + pr
            ev = ss < LEN_EVEN

            @pl.when(g > 0)
            def _():
                out_copy(par, 0).wait()

            for b in range(1, NSEQ, 2):
                in_copy(par, b, _X_OFF[b] + ss).start()

            @pl.when(ev)
            def _():
                for b in range(0, NSEQ, 2):
                    in_copy(par, b, _X_OFF[b] + ss).start()

            @pl.when((ss >= ev_z) & (ss < ev_z + 2 * SROWS))
            def _zero_even():
                def zj(j, _):
                    sl = pl.ds(j * LANE, LANE)
                    for s in range(SROWS):
                        for b in range(0, NSEQ, 2):
                            obs[par][s, b, sl] = zero
                    return 0

                lax.fori_loop(0, VPR, zj, 0)

            @pl.when(ev)
            def _():
                pltpu.make_async_copy(
                    out_hbm.at[pl.ds(0, SROWS)], obs[par],
                    in_sems[par]).wait()

            @pl.when(jnp.logical_not(ev))
            def _():
                pltpu.make_async_copy(
                    out_hbm.at[pl.ds(0, SROWS), pl.ds(0, NSEQ // 2)],
                    obs[par].at[:, pl.ds(0, NSEQ // 2)],
                    in_sems[par]).wait()

            @pl.when(ev)
            def _add_all():
                def aj(j, _):
                    sl = pl.ds(j * LANE, LANE)
                    for s in range(SROWS):
                        pv = pe_buf[pr + s, 0, sl]
                        for b in range(NSEQ):
                            obs[par][s, b, sl] = obs[par][s, b, sl] + pv
                    return 0

                lax.fori_loop(0, VPR, aj, 0)

            @pl.when(jnp.logical_not(ev))
            def _add_odd():
                def aj(j, _):
                    sl = pl.ds(j * LANE, LANE)
                    for s in range(SROWS):
                        pv = pe_buf[pr + s, 0, sl]
                        for b in range(1, NSEQ, 2):
                            obs[par][s, b, sl] = obs[par][s, b, sl] + pv
                    return 0

                lax.fori_loop(0, VPR, aj, 0)

            out_copy(par, ss).start()

        return 0

    lax.fori_loop(0, NSLAB // 2, slab, 0)

    out_copy(0, 0).wait()
    out_copy(1, 0).wait()


_pe_pad_kernel = functools.partial(
    pl.kernel,
    out_type=jax.ShapeDtypeStruct((MAXLEN, NSEQ, EMB), jnp.float32),
    mesh=plsc.VectorSubcoreMesh(core_axis_name="c", subcore_axis_name="s",
                                num_cores=2, num_subcores=16),
    scratch_types=[
        pltpu.VMEM((S_PER_W, 1, EMB), jnp.float32),
        pltpu.VMEM((SROWS, NSEQ, EMB), jnp.float32),
        pltpu.VMEM((SROWS, NSEQ, EMB), jnp.float32),
        pltpu.SemaphoreType.DMA, pltpu.SemaphoreType.DMA,
        pltpu.SemaphoreType.DMA, pltpu.SemaphoreType.DMA,
    ],
)(_pe_pad_body)


def kernel(x, length, pe):
    del length
    x3 = x.reshape(TOTAL, 1, EMB)
    return _pe_pad_kernel(x3, pe)


# speedup vs baseline: 2.5015x; 1.1978x over previous
import functools

import jax
import jax.numpy as jnp
from jax import lax
from jax.experimental import pallas as pl
from jax.experimental.pallas import tpu as pltpu
from jax.experimental.pallas import tpu_sc as plsc

EMB = 512
NSEQ = 16
MAXLEN = 3072
TOTAL = 32768
LEN_EVEN = 1024
PAIR = 4096

NW = 32
S_PER_W = MAXLEN // NW
SROWS = 4
NSLAB = S_PER_W // SROWS
NBUF = 3
LANE = 16
VPR = EMB // LANE

_X_OFF = [(b >> 1) * PAIR + (b & 1) * LEN_EVEN for b in range(NSEQ)]
_SLAB_B = SROWS * EMB * 4


def _pe_pad_body(x_hbm, pe_hbm, out_hbm,
                 ob0, ob1, ob2, pb0, pb1, pb2,
                 si0, si1, si2, so0, so1, so2):
    wid = lax.axis_index("s") * 2 + lax.axis_index("c")
    s0 = wid * S_PER_W

    obs = (ob0, ob1, ob2)
    pbs = (pb0, pb1, pb2)
    in_sems = (si0, si1, si2)
    out_sems = (so0, so1, so2)

    zero = jnp.zeros((LANE,), jnp.float32)
    ev_z = jnp.maximum(s0, LEN_EVEN)

    def out_copy(par, ss):
        return pltpu.make_async_copy(
            obs[par], out_hbm.at[pl.ds(ss, SROWS)], out_sems[par])

    def fire_reads(par, k):
        ss = s0 + k * SROWS
        pltpu.make_async_copy(
            pe_hbm.at[pl.ds(ss, SROWS)], pbs[par], in_sems[par]).start()
        for b in range(1, NSEQ, 2):
            pltpu.make_async_copy(
                x_hbm.at[pl.ds(_X_OFF[b] + ss, SROWS)],
                obs[par].at[:, pl.ds(b, 1)], in_sems[par]).start()

        @pl.when(ss < LEN_EVEN)
        def _():
            for b in range(0, NSEQ, 2):
                pltpu.make_async_copy(
                    x_hbm.at[pl.ds(_X_OFF[b] + ss, SROWS)],
                    obs[par].at[:, pl.ds(b, 1)], in_sems[par]).start()

    fire_reads(0, 0)

    def group(g, _):
        for r in range(NBUF):
            k = NBUF * g + r
            ss = s0 + k * SROWS
            ev = ss < LEN_EVEN

            r1 = (r + 1) % NBUF

            @pl.when(k + 1 < NSLAB)
            def _():
                @pl.when(k + 1 >= NBUF)
                def _():
                    out_copy(r1, 0).wait()

                fire_reads(r1, k + 1)

            @pl.when((ss >= ev_z) & (ss < ev_z + NBUF * SROWS))
            def _zero_even():
                def zj(j, _):
                    sl = pl.ds(j * LANE, LANE)
                    for s in range(SROWS):
                        for b in range(0, NSEQ, 2):
                            obs[r][s, b, sl] = zero
                    return 0

                lax.fori_loop(0, VPR, zj, 0)

            pltpu.make_async_copy(
                pe_hbm.at[pl.ds(0, SROWS)], pbs[r], in_sems[r]).wait()

            @pl.when(ev)
            def _():
                pltpu.make_async_copy(
                    out_hbm.at[pl.ds(0, SROWS)], obs[r], in_sems[r]).wait()

            @pl.when(jnp.logical_not(ev))
            def _():
                pltpu.make_async_copy(
                    out_hbm.at[pl.ds(0, SROWS), pl.ds(0, NSEQ // 2)],
                    obs[r].at[:, pl.ds(0, NSEQ // 2)], in_sems[r]).wait()

            @pl.when(ev)
            def _add_all():
                def aj(j, _):
                    sl = pl.ds(j * LANE, LANE)
                    for s in range(SROWS):
                        pv = pbs[r][s, 0, sl]
                        for b in range(NSEQ):
                            obs[r][s, b, sl] = obs[r][s, b, sl] + pv
                    return 0

                lax.fori_loop(0, VPR, aj, 0)

            @pl.when(jnp.logical_not(ev))
            def _add_odd():
                def aj(j, _):
                    sl = pl.ds(j * LANE, LANE)
                    for s in range(SROWS):
                        pv = pbs[r][s, 0, sl]
                        for b in range(1, NSEQ, 2):
                            obs[r][s, b, sl] = obs[r][s, b, sl] + pv
                    return 0

                lax.fori_loop(0, VPR, aj, 0)

            out_copy(r, ss).start()

        return 0

    lax.fori_loop(0, NSLAB // NBUF, group, 0)

    for r in range(NBUF):
        out_copy(r, 0).wait()


_pe_pad_kernel = functools.partial(
    pl.kernel,
    out_type=jax.ShapeDtypeStruct((MAXLEN, NSEQ, EMB), jnp.float32),
    mesh=plsc.VectorSubcoreMesh(core_axis_name="c", subcore_axis_name="s",
                                num_cores=2, num_subcores=16),
    scratch_types=[
        pltpu.VMEM((SROWS, NSEQ, EMB), jnp.float32),
        pltpu.VMEM((SROWS, NSEQ, EMB), jnp.float32),
        pltpu.VMEM((SROWS, NSEQ, EMB), jnp.float32),
        pltpu.VMEM((SROWS, 1, EMB), jnp.float32),
        pltpu.VMEM((SROWS, 1, EMB), jnp.float32),
        pltpu.VMEM((SROWS, 1, EMB), jnp.float32),
        pltpu.SemaphoreType.DMA, pltpu.SemaphoreType.DMA, pltpu.SemaphoreType.DMA,
        pltpu.SemaphoreType.DMA, pltpu.SemaphoreType.DMA, pltpu.SemaphoreType.DMA,
    ],
)(_pe_pad_body)


def kernel(x, length, pe):
    del length
    x3 = x.reshape(TOTAL, 1, EMB)
    return _pe_pad_kernel(x3, pe)
